# baseline (speedup 1.0000x reference)
import functools

import jax
import jax.numpy as jnp
from jax import lax
from jax.experimental import pallas as pl
from jax.experimental.pallas import tpu as pltpu
from jax.experimental.pallas import tpu_sc as plsc

N_NODES = 10000
D = 128
N_EDGES = 320000
NC = 2
NS = 16
NW = NC * NS
EPW = N_EDGES // NW
C = 80
N_CHUNKS = EPW // C


def _normalize_body(z_ref, o_ref):
    x = z_ref[...]
    ss = jnp.sum(x * x, axis=1, keepdims=True)
    o_ref[...] = x * lax.rsqrt(ss)


def _normalize(z):
    return pl.pallas_call(
        _normalize_body,
        out_shape=jax.ShapeDtypeStruct((N_NODES, D), jnp.float32),
    )(z)


_SC_MESH = plsc.VectorSubcoreMesh(core_axis_name="c", subcore_axis_name="s")


@functools.partial(
    pl.kernel,
    out_type=jax.ShapeDtypeStruct((N_EDGES,), jnp.float32),
    mesh=_SC_MESH,
    scratch_types=[
        pltpu.VMEM((EPW,), jnp.int32),
        pltpu.VMEM((EPW,), jnp.int32),
        pltpu.VMEM((2, C, D), jnp.float32),
        pltpu.VMEM((2, C, D), jnp.float32),
        pltpu.VMEM((C,), jnp.float32),
        pltpu.SemaphoreType.DMA,
        pltpu.SemaphoreType.DMA,
        pltpu.SemaphoreType.DMA,
        pltpu.SemaphoreType.DMA,
    ],
)
def _sc_edge_dot(zn_hbm, src_hbm, dst_hbm, out_hbm,
                 idx_s, idx_d, rows_s, rows_d, out_v,
                 sem_s0, sem_d0, sem_s1, sem_d1):
    wid = lax.axis_index("s") * NC + lax.axis_index("c")
    wbase = wid * EPW
    pltpu.sync_copy(src_hbm.at[pl.ds(wbase, EPW)], idx_s)
    pltpu.sync_copy(dst_hbm.at[pl.ds(wbase, EPW)], idx_d)

    sems = ((sem_s0, sem_d0), (sem_s1, sem_d1))

    def issue(buf, i):
        off = i * C
        sem_s, sem_d = sems[buf]
        pltpu.async_copy(
            zn_hbm.at[idx_s.at[pl.ds(off, C)]], rows_s.at[buf], sem_s)
        pltpu.async_copy(
            zn_hbm.at[idx_d.at[pl.ds(off, C)]], rows_d.at[buf], sem_d)

    def wait(buf):
        sem_s, sem_d = sems[buf]
        pltpu.make_async_copy(
            zn_hbm.at[idx_s.at[pl.ds(0, C)]], rows_s.at[buf], sem_s).wait()
        pltpu.make_async_copy(
            zn_hbm.at[idx_d.at[pl.ds(0, C)]], rows_d.at[buf], sem_d).wait()

    lane = lax.iota(jnp.int32, 16)
    _dnums = lax.GatherDimensionNumbers(
        offset_dims=(), collapsed_slice_dims=(0,), start_index_map=(0,))

    def _permute(x, idx):
        return lax.gather(x, idx[:, None], _dnums, slice_sizes=(1,),
                          mode=lax.GatherScatterMode.PROMISE_IN_BOUNDS)

    xperms = {r: lane ^ r for r in (8, 4, 2, 1)}
    masks = {r: (lane & r) == 0 for r in (8, 4, 2, 1)}
    BITREV = (0, 8, 4, 12, 2, 10, 6, 14, 1, 9, 5, 13, 3, 11, 7, 15)

    def group_loop(buf, out_off):
        rs = rows_s.at[buf]
        rd = rows_d.at[buf]

        for k in range(C // 16):
            base = k * 16
            vs = []
            for s in range(16):
                e = base + BITREV[s]
                ps = [rs[e, pl.ds(j * 16, 16)] * rd[e, pl.ds(j * 16, 16)]
                      for j in range(D // 16)]
                while len(ps) > 1:
                    ps = [a + b for a, b in zip(ps[::2], ps[1::2])]
                vs.append(ps[0])
            for r in (8, 4, 2, 1):
                vs = [jnp.where(masks[r],
                                x + _permute(x, xperms[r]),
                                y + _permute(y, xperms[r]))
                      for x, y in zip(vs[::2], vs[1::2])]
            out_v[pl.ds(base, 16)] = vs[0] * 0.5 + 0.5

        pltpu.sync_copy(out_v, out_hbm.at[pl.ds(wbase + out_off, C)])

    issue(0, 0)

    def pair_body(p, _):
        i0 = 2 * p
        issue(1, i0 + 1)
        wait(0)
        group_loop(0, i0 * C)
        issue(0, i0 + 2)
        wait(1)
        group_loop(1, (i0 + 1) * C)
        return 0

    lax.fori_loop(0, (N_CHUNKS - 1) // 2, pair_body, 0)
    wait(0)
    group_loop(0, (N_CHUNKS - 1) * C)


def kernel(z, edge_index):
    zn = _normalize(z)
    ei = edge_index.astype(jnp.int32)
    return _sc_edge_dot(zn, ei[0], ei[1])


# speedup vs baseline: 3.2453x; 3.2453x over previous
import functools

import jax
import jax.numpy as jnp
from jax import lax
from jax.experimental import pallas as pl
from jax.experimental.pallas import tpu as pltpu
from jax.experimental.pallas import tpu_sc as plsc

N_NODES = 10000
D = 128
N_EDGES = 320000
NC = 2
NS = 16
NW = NC * NS
EPW = N_EDGES // NW
C = 80
N_CHUNKS = EPW // C


def _normalize_body(z_ref, o_ref):
    x = z_ref[...]
    ss = jnp.sum(x * x, axis=1, keepdims=True)
    o_ref[...] = (x * lax.rsqrt(ss)).astype(jnp.bfloat16)


def _normalize(z):
    zn = pl.pallas_call(
        _normalize_body,
        out_shape=jax.ShapeDtypeStruct((N_NODES, D), jnp.bfloat16),
    )(z)
    return lax.bitcast_convert_type(zn.reshape(N_NODES, D // 2, 2), jnp.int32)


_SC_MESH = plsc.VectorSubcoreMesh(core_axis_name="c", subcore_axis_name="s")


@functools.partial(
    pl.kernel,
    out_type=jax.ShapeDtypeStruct((N_EDGES,), jnp.float32),
    mesh=_SC_MESH,
    compiler_params=pltpu.CompilerParams(
        use_tc_tiling_on_sc=False, needs_layout_passes=False),
    scratch_types=[
        pltpu.VMEM((EPW,), jnp.int32),
        pltpu.VMEM((EPW,), jnp.int32),
        pltpu.VMEM((2, C, D // 2), jnp.int32),
        pltpu.VMEM((2, C, D // 2), jnp.int32),
        pltpu.VMEM((C,), jnp.float32),
        pltpu.SemaphoreType.DMA,
        pltpu.SemaphoreType.DMA,
        pltpu.SemaphoreType.DMA,
        pltpu.SemaphoreType.DMA,
    ],
)
def _sc_edge_dot(zn_hbm, src_hbm, dst_hbm, out_hbm,
                 idx_s, idx_d, rows_s, rows_d, out_v,
                 sem_s0, sem_d0, sem_s1, sem_d1):
    wid = lax.axis_index("s") * NC + lax.axis_index("c")
    wbase = wid * EPW
    pltpu.sync_copy(src_hbm.at[pl.ds(wbase, EPW)], idx_s)
    pltpu.sync_copy(dst_hbm.at[pl.ds(wbase, EPW)], idx_d)

    sems = ((sem_s0, sem_d0), (sem_s1, sem_d1))

    def issue(buf, i):
        off = i * C
        sem_s, sem_d = sems[buf]
        pltpu.async_copy(
            zn_hbm.at[idx_s.at[pl.ds(off, C)]], rows_s.at[buf], sem_s)
        pltpu.async_copy(
            zn_hbm.at[idx_d.at[pl.ds(off, C)]], rows_d.at[buf], sem_d)

    def wait(buf):
        sem_s, sem_d = sems[buf]
        pltpu.make_async_copy(
            zn_hbm.at[idx_s.at[pl.ds(0, C)]], rows_s.at[buf], sem_s).wait()
        pltpu.make_async_copy(
            zn_hbm.at[idx_d.at[pl.ds(0, C)]], rows_d.at[buf], sem_d).wait()

    lane = lax.iota(jnp.int32, 16)
    rots = [(lane + r) % 16 for r in (8, 4, 2, 1)]
    _dnums = lax.GatherDimensionNumbers(
        offset_dims=(), collapsed_slice_dims=(0,), start_index_map=(0,))

    def _permute(x, idx):
        return lax.gather(x, idx[:, None], _dnums, slice_sizes=(1,),
                          mode=lax.GatherScatterMode.PROMISE_IN_BOUNDS)

    mask_hi = jnp.full((16,), -65536, jnp.int32)

    def group_loop(buf, out_off):
        rs = rows_s.at[buf]
        rd = rows_d.at[buf]

        def group_body(k, _):
            base = k * 16
            vec = jnp.zeros((16,), jnp.float32)
            for g in range(16):
                e = base + g
                acc = None
                for j in range(D // 32):
                    xs = rs[e, pl.ds(j * 16, 16)]
                    xd = rd[e, pl.ds(j * 16, 16)]
                    s_lo = plsc.bitcast(xs << 16, jnp.float32)
                    s_hi = plsc.bitcast(xs & mask_hi, jnp.float32)
                    d_lo = plsc.bitcast(xd << 16, jnp.float32)
                    d_hi = plsc.bitcast(xd & mask_hi, jnp.float32)
                    p = s_lo * d_lo + s_hi * d_hi
                    acc = p if acc is None else acc + p
                for rot in rots:
                    acc = acc + _permute(acc, rot)
                vec = jnp.where(lane == g, acc * 0.5 + 0.5, vec)
            out_v[pl.ds(base, 16)] = vec
            return 0

        lax.fori_loop(0, C // 16, group_body, 0)
        pltpu.sync_copy(out_v, out_hbm.at[pl.ds(wbase + out_off, C)])

    issue(0, 0)

    def pair_body(p, _):
        i0 = 2 * p
        issue(1, i0 + 1)
        wait(0)
        group_loop(0, i0 * C)
        issue(0, i0 + 2)
        wait(1)
        group_loop(1, (i0 + 1) * C)
        return 0

    lax.fori_loop(0, (N_CHUNKS - 1) // 2, pair_body, 0)
    wait(0)
    group_loop(0, (N_CHUNKS - 1) * C)


def kernel(z, edge_index):
    zn = _normalize(z)
    ei = edge_index.astype(jnp.int32)
    return _sc_edge_dot(zn, ei[0], ei[1])


# speedup vs baseline: 3.4813x; 1.0727x over previous
import functools

import jax
import jax.numpy as jnp
from jax import lax
from jax.experimental import pallas as pl
from jax.experimental.pallas import tpu as pltpu
from jax.experimental.pallas import tpu_sc as plsc

N_NODES = 10000
D = 128
N_EDGES = 320000
NC = 2
NS = 16
NW = NC * NS
EPW = N_EDGES // NW
C = 80
N_CHUNKS = EPW // C


def _normalize_body(z_ref, o_ref):
    x = z_ref[...]
    ss = jnp.sum(x * x, axis=1, keepdims=True)
    o_ref[...] = (x * lax.rsqrt(ss)).astype(jnp.bfloat16)


def _normalize(z):
    zn = pl.pallas_call(
        _normalize_body,
        out_shape=jax.ShapeDtypeStruct((N_NODES, D), jnp.bfloat16),
    )(z)
    return lax.bitcast_convert_type(zn.reshape(N_NODES, D // 2, 2), jnp.int32)


_SC_MESH = plsc.VectorSubcoreMesh(core_axis_name="c", subcore_axis_name="s")


@functools.partial(
    pl.kernel,
    out_type=jax.ShapeDtypeStruct((N_EDGES,), jnp.float32),
    mesh=_SC_MESH,
    compiler_params=pltpu.CompilerParams(
        use_tc_tiling_on_sc=False, needs_layout_passes=False),
    scratch_types=[
        pltpu.VMEM((EPW,), jnp.int32),
        pltpu.VMEM((EPW,), jnp.int32),
        pltpu.VMEM((2, C, D // 2), jnp.int32),
        pltpu.VMEM((2, C, D // 2), jnp.int32),
        pltpu.VMEM((C,), jnp.float32),
        pltpu.VMEM_SHARED((N_NODES, D // 2), jnp.int32),
        pltpu.SemaphoreType.DMA,
        pltpu.SemaphoreType.DMA,
        pltpu.SemaphoreType.DMA,
        pltpu.SemaphoreType.DMA,
    ],
)
def _sc_edge_dot(zn_hbm, src_hbm, dst_hbm, out_hbm,
                 idx_s, idx_d, rows_s, rows_d, out_v, table,
                 sem_s0, sem_d0, sem_s1, sem_d1):
    sid = lax.axis_index("s")
    wid = sid * NC + lax.axis_index("c")
    wbase = wid * EPW
    slab = N_NODES // NS
    pltpu.sync_copy(zn_hbm.at[pl.ds(sid * slab, slab)],
                    table.at[pl.ds(sid * slab, slab)])
    pltpu.sync_copy(src_hbm.at[pl.ds(wbase, EPW)], idx_s)
    pltpu.sync_copy(dst_hbm.at[pl.ds(wbase, EPW)], idx_d)
    plsc.subcore_barrier()

    sems = ((sem_s0, sem_d0), (sem_s1, sem_d1))

    def issue(buf, i):
        off = i * C
        sem_s, sem_d = sems[buf]
        pltpu.async_copy(
            table.at[idx_s.at[pl.ds(off, C)]], rows_s.at[buf], sem_s)
        pltpu.async_copy(
            table.at[idx_d.at[pl.ds(off, C)]], rows_d.at[buf], sem_d)

    def wait(buf):
        sem_s, sem_d = sems[buf]
        pltpu.make_async_copy(
            table.at[idx_s.at[pl.ds(0, C)]], rows_s.at[buf], sem_s).wait()
        pltpu.make_async_copy(
            table.at[idx_d.at[pl.ds(0, C)]], rows_d.at[buf], sem_d).wait()

    lane = lax.iota(jnp.int32, 16)
    rots = [(lane + r) % 16 for r in (8, 4, 2, 1)]
    _dnums = lax.GatherDimensionNumbers(
        offset_dims=(), collapsed_slice_dims=(0,), start_index_map=(0,))

    def _permute(x, idx):
        return lax.gather(x, idx[:, None], _dnums, slice_sizes=(1,),
                          mode=lax.GatherScatterMode.PROMISE_IN_BOUNDS)

    mask_hi = jnp.full((16,), -65536, jnp.int32)

    def group_loop(buf, out_off):
        rs = rows_s.at[buf]
        rd = rows_d.at[buf]

        def group_body(k, _):
            base = k * 16
            vec = jnp.zeros((16,), jnp.float32)
            for g in range(16):
                e = base + g
                acc = None
                for j in range(D // 32):
                    xs = rs[e, pl.ds(j * 16, 16)]
                    xd = rd[e, pl.ds(j * 16, 16)]
                    s_lo = plsc.bitcast(xs << 16, jnp.float32)
                    s_hi = plsc.bitcast(xs & mask_hi, jnp.float32)
                    d_lo = plsc.bitcast(xd << 16, jnp.float32)
                    d_hi = plsc.bitcast(xd & mask_hi, jnp.float32)
                    p = s_lo * d_lo + s_hi * d_hi
                    acc = p if acc is None else acc + p
                for rot in rots:
                    acc = acc + _permute(acc, rot)
                vec = jnp.where(lane == g, acc * 0.5 + 0.5, vec)
            out_v[pl.ds(base, 16)] = vec
            return 0

        lax.fori_loop(0, C // 16, group_body, 0)
        pltpu.sync_copy(out_v, out_hbm.at[pl.ds(wbase + out_off, C)])

    issue(0, 0)

    def pair_body(p, _):
        i0 = 2 * p
        issue(1, i0 + 1)
        wait(0)
        group_loop(0, i0 * C)
        issue(0, i0 + 2)
        wait(1)
        group_loop(1, (i0 + 1) * C)
        return 0

    lax.fori_loop(0, (N_CHUNKS - 1) // 2, pair_body, 0)
    wait(0)
    group_loop(0, (N_CHUNKS - 1) * C)


def kernel(z, edge_index):
    zn = _normalize(z)
    ei = edge_index.astype(jnp.int32)
    return _sc_edge_dot(zn, ei[0], ei[1])


# speedup vs baseline: 3.5363x; 1.0158x over previous
import functools

import jax
import jax.numpy as jnp
from jax import lax
from jax.experimental import pallas as pl
from jax.experimental.pallas import tpu as pltpu
from jax.experimental.pallas import tpu_sc as plsc

N_NODES = 10000
D = 128
N_EDGES = 320000
NC = 2
NS = 16
NW = NC * NS
EPW = N_EDGES // NW
C = 400
SUB = 80
NSUB = C // SUB
N_CHUNKS = EPW // C


def _normalize_body(z_ref, o_ref):
    x = z_ref[...]
    ss = jnp.sum(x * x, axis=1, keepdims=True)
    o_ref[...] = (x * lax.rsqrt(ss)).astype(jnp.bfloat16)


def _normalize(z):
    zn = pl.pallas_call(
        _normalize_body,
        out_shape=jax.ShapeDtypeStruct((N_NODES, D), jnp.bfloat16),
    )(z)
    return lax.bitcast_convert_type(zn.reshape(N_NODES, D // 2, 2), jnp.int32)


_SC_MESH = plsc.VectorSubcoreMesh(core_axis_name="c", subcore_axis_name="s")


@functools.partial(
    pl.kernel,
    out_type=jax.ShapeDtypeStruct((N_EDGES,), jnp.float32),
    mesh=_SC_MESH,
    compiler_params=pltpu.CompilerParams(
        use_tc_tiling_on_sc=False, needs_layout_passes=False),
    scratch_types=[
        pltpu.VMEM((EPW,), jnp.int32),
        pltpu.VMEM((EPW,), jnp.int32),
        pltpu.VMEM((2, C, D // 2), jnp.int32),
        pltpu.VMEM((2, C, D // 2), jnp.int32),
        pltpu.VMEM((C,), jnp.float32),
        pltpu.SemaphoreType.DMA,
        pltpu.SemaphoreType.DMA,
        pltpu.SemaphoreType.DMA,
        pltpu.SemaphoreType.DMA,
    ],
)
def _sc_edge_dot(zn_hbm, src_hbm, dst_hbm, out_hbm,
                 idx_s, idx_d, rows_s, rows_d, out_v,
                 sem_s0, sem_d0, sem_s1, sem_d1):
    sid = lax.axis_index("s")
    wid = sid * NC + lax.axis_index("c")
    wbase = wid * EPW
    pltpu.sync_copy(src_hbm.at[pl.ds(wbase, EPW)], idx_s)
    pltpu.sync_copy(dst_hbm.at[pl.ds(wbase, EPW)], idx_d)

    sems = ((sem_s0, sem_d0), (sem_s1, sem_d1))

    def issue(buf, i):
        off = i * C
        sem_s, sem_d = sems[buf]
        for k in range(NSUB):
            pltpu.async_copy(
                zn_hbm.at[idx_s.at[pl.ds(off + k * SUB, SUB)]],
                rows_s.at[buf, pl.ds(k * SUB, SUB)], sem_s)
            pltpu.async_copy(
                zn_hbm.at[idx_d.at[pl.ds(off + k * SUB, SUB)]],
                rows_d.at[buf, pl.ds(k * SUB, SUB)], sem_d)

    def wait(buf):
        sem_s, sem_d = sems[buf]
        for k in range(NSUB):
            pltpu.make_async_copy(
                zn_hbm.at[idx_s.at[pl.ds(0, SUB)]],
                rows_s.at[buf, pl.ds(0, SUB)], sem_s).wait()
            pltpu.make_async_copy(
                zn_hbm.at[idx_d.at[pl.ds(0, SUB)]],
                rows_d.at[buf, pl.ds(0, SUB)], sem_d).wait()

    lane = lax.iota(jnp.int32, 16)
    rots = [(lane + r) % 16 for r in (8, 4, 2, 1)]
    _dnums = lax.GatherDimensionNumbers(
        offset_dims=(), collapsed_slice_dims=(0,), start_index_map=(0,))

    def _permute(x, idx):
        return lax.gather(x, idx[:, None], _dnums, slice_sizes=(1,),
                          mode=lax.GatherScatterMode.PROMISE_IN_BOUNDS)

    mask_hi = jnp.full((16,), -65536, jnp.int32)

    def group_loop(buf, out_off):
        rs = rows_s.at[buf]
        rd = rows_d.at[buf]

        def group_body(k, _):
            base = k * 16
            vec = jnp.zeros((16,), jnp.float32)
            for g in range(16):
                e = base + g
                acc = None
                for j in range(D // 32):
                    xs = rs[e, pl.ds(j * 16, 16)]
                    xd = rd[e, pl.ds(j * 16, 16)]
                    s_lo = plsc.bitcast(xs << 16, jnp.float32)
                    s_hi = plsc.bitcast(xs & mask_hi, jnp.float32)
                    d_lo = plsc.bitcast(xd << 16, jnp.float32)
                    d_hi = plsc.bitcast(xd & mask_hi, jnp.float32)
                    p = s_lo * d_lo + s_hi * d_hi
                    acc = p if acc is None else acc + p
                for rot in rots:
                    acc = acc + _permute(acc, rot)
                vec = jnp.where(lane == g, acc * 0.5 + 0.5, vec)
            out_v[pl.ds(base, 16)] = vec
            return 0

        lax.fori_loop(0, C // 16, group_body, 0)
        pltpu.sync_copy(out_v, out_hbm.at[pl.ds(wbase + out_off, C)])

    issue(0, 0)

    def pair_body(p, _):
        i0 = 2 * p
        issue(1, i0 + 1)
        wait(0)
        group_loop(0, i0 * C)
        issue(0, i0 + 2)
        wait(1)
        group_loop(1, (i0 + 1) * C)
        return 0

    lax.fori_loop(0, (N_CHUNKS - 1) // 2, pair_body, 0)
    wait(0)
    group_loop(0, (N_CHUNKS - 1) * C)


def kernel(z, edge_index):
    zn = _normalize(z)
    ei = edge_index.astype(jnp.int32)
    return _sc_edge_dot(zn, ei[0], ei[1])
